# Initial kernel scaffold; baseline (speedup 1.0000x reference)
import jax
import jax.numpy as jnp
from jax.experimental import pallas as pl


def kernel(objs, triples, boxes_gt, angles_gt, attributes, obj_to_img, params):
    raise NotImplementedError("write your pallas kernel here")


# speedup vs baseline: 2.1834x; 2.1834x over previous
import functools

import jax
import jax.numpy as jnp
from jax import lax
from jax.experimental import pallas as pl
from jax.experimental.pallas import tpu as pltpu
from jax.experimental.pallas import tpu_sc as plsc

F32 = jnp.float32
I32 = jnp.int32

O_N = 10000
T_N = 160000
NO = 36
NP = 16
NA = 8
NANG = 24
HID = 512

NC, NS = 2, 16
NW = NC * NS
PER_W = T_N // NW
CB = 40
NCH = PER_W // CB
O_PAD = 10240
STR = O_PAD // NS

TT = 2000
GT = T_N // TT
GO = O_N // TT


def _wspec(a):
    return pl.BlockSpec(a.shape, lambda i: (0,) * a.ndim)


def _espec(d):
    return pl.BlockSpec((TT, d), lambda i: (i, 0))


def _idxspec():
    return pl.BlockSpec((1, 1, TT), lambda i: (i, 0, 0))


def _onehot(v, n):
    return (v[:, None] == lax.broadcasted_iota(I32, (TT, n), 1)).astype(F32)


def _dot(a, b):
    return jnp.dot(a, b, preferred_element_type=F32)


def _prep_body(objs_r, attrs_r, angs_r, boxes_r, oec, aec, ange, wbT, bb,
               odc, adc, pec, a1pTe, pdc, a1pTd,
               ove_r, ovd_r, attrd_r, pre1e_r, pre1d_r):
    oh_obj = _onehot(objs_r[0, 0], NO)
    oh_att = _onehot(attrs_r[0, 0], NA)
    e_obj = _dot(oh_obj, oec[...])
    e_att = _dot(oh_att, aec[...])
    e_box = _dot(boxes_r[...], wbT[...]) + bb[...]
    e_ang = _dot(_onehot(angs_r[0, 0], NANG), ange[...])
    ove_r[...] = jnp.concatenate([e_obj, e_att, e_box, e_ang], axis=1)
    d_obj = _dot(oh_obj, odc[...])
    d_att = _dot(oh_att, adc[...])
    ovd_r[...] = jnp.concatenate([d_obj, d_att], axis=1)
    attrd_r[...] = d_att
    pre1e_r[...] = _dot(pec[...], a1pTe[...])
    pre1d_r[...] = _dot(pdc[...], a1pTd[...])


def _prep_call(objs3, attrs3, angs3, boxes, params, a1pTe, a1pTd):
    wb, bb = params['box_embeddings']
    ins = [objs3, attrs3, angs3, boxes,
           params['obj_emb_ec'], params['attr_emb_ec'], params['angle_emb'],
           wb.T, bb.reshape(1, -1),
           params['obj_emb_dc'], params['attr_emb_dc'],
           params['pred_emb_ec'], a1pTe, params['pred_emb_dc'], a1pTd]
    specs = [_idxspec(), _idxspec(), _idxspec(),
             pl.BlockSpec((TT, 6), lambda i: (i, 0))] + [_wspec(x) for x in ins[4:]]
    out_shape = [jax.ShapeDtypeStruct((O_N, 256), F32),
                 jax.ShapeDtypeStruct((O_N, 128), F32),
                 jax.ShapeDtypeStruct((O_N, 32), F32),
                 jax.ShapeDtypeStruct((NP, HID), F32),
                 jax.ShapeDtypeStruct((NP, HID), F32)]
    out_specs = [_espec(256), _espec(128), _espec(32),
                 pl.BlockSpec((NP, HID), lambda i: (0, 0)),
                 pl.BlockSpec((NP, HID), lambda i: (0, 0))]
    return pl.pallas_call(_prep_body, grid=(GO,), in_specs=specs,
                          out_specs=out_specs, out_shape=out_shape)(*ins)


def _edge_body(layer1, last):
    def body(*refs):
        if layer1:
            pv, gs, go, pmat, a1sT, a1oT, b1, w2sT, b2s, w2oT, b2o, *rest = refs
        else:
            pv, gs, go, pmat, a1sT, a1oT, b1, w2sT, b2s, w2oT, b2o, *rest = refs
        if not last:
            w2pT, b2p, *rest = rest
        outs = rest
        h = _dot(gs[...], a1sT[...]) + _dot(go[...], a1oT[...])
        if layer1:
            h += _dot(_onehot(pv[0, 0], NP), pmat[...])
        else:
            h += _dot(pv[...], pmat[...])
        h = jnp.maximum(h + b1[...], 0.0)
        s = jnp.maximum(_dot(h, w2sT[...]) + b2s[...], 0.0)
        o = jnp.maximum(_dot(h, w2oT[...]) + b2o[...], 0.0)
        for k in range(4):
            outs[k][...] = s[:, k * 128:(k + 1) * 128]
            outs[4 + k][...] = o[:, k * 128:(k + 1) * 128]
        if not last:
            outs[8][...] = jnp.maximum(_dot(h, w2pT[...]) + b2p[...], 0.0)
    return body


def _edge_layer(gs, go, pvp, w, layer1, last, din):
    if layer1:
        ins = [pvp, gs, go, w['pre1']]
        specs = [_idxspec(), _espec(din), _espec(din), _wspec(w['pre1'])]
    else:
        ins = [pvp, gs, go, w['a1pT']]
        specs = [_espec(din), _espec(din), _espec(din), _wspec(w['a1pT'])]
    tail = [w['a1sT'], w['a1oT'], w['b1'], w['w2sT'], w['b2s'], w['w2oT'], w['b2o']]
    if not last:
        tail += [w['w2pT'], w['b2p']]
    ins += tail
    specs += [_wspec(x) for x in tail]
    out_shape = [jax.ShapeDtypeStruct((T_N, 128), F32) for _ in range(8)]
    out_specs = [_espec(128) for _ in range(8)]
    if not last:
        out_shape.append(jax.ShapeDtypeStruct((T_N, din), F32))
        out_specs.append(_espec(din))
    return pl.pallas_call(_edge_body(layer1, last), grid=(GT,), in_specs=specs,
                          out_specs=out_specs, out_shape=out_shape)(*ins)


def _net2_body(p_r, c_r, aT, ab, bT, bb, out_r):
    cnt = c_r[0, :, 0:1] + c_r[1, :, 0:1]
    inv = 1.0 / jnp.maximum(cnt, 1.0)
    aTm = aT[...]
    h = jnp.zeros((TT, HID), F32) + ab[...]
    for k in range(4):
        pk = (p_r[0, k] + p_r[1, k]) * inv
        h += _dot(pk, aTm[k * 128:(k + 1) * 128])
    h = jnp.maximum(h, 0.0)
    out_r[...] = jnp.maximum(_dot(h, bT[...]) + bb[...], 0.0)


def _net2_layer(parts, counts, w, dn):
    ins = [parts, counts, w['n2aT'], w['n2b1'], w['n2bT'], w['n2b2']]
    specs = [pl.BlockSpec((NC, 4, TT, 128), lambda i: (0, 0, i, 0)),
             pl.BlockSpec((NC, TT, 128), lambda i: (0, i, 0)),
             _wspec(w['n2aT']), _wspec(w['n2b1']),
             _wspec(w['n2bT']), _wspec(w['n2b2'])]
    return pl.pallas_call(_net2_body, grid=(GO,), in_specs=specs,
                          out_specs=_espec(dn),
                          out_shape=jax.ShapeDtypeStruct((O_N, dn), F32))(*ins)


def _heads_enc_body(x_r, m1T, m1b, m2T, m2b, bmT, bmb, bvT, bvb,
                    a1T, a1b, a2T, a2b, amT, amb, avT, avb, mu_r, lv_r):
    x = x_r[...]
    ovb = jnp.maximum(_dot(x, m1T[...]) + m1b[...], 0.0)
    ovb = jnp.maximum(_dot(ovb, m2T[...]) + m2b[...], 0.0)
    mub = _dot(ovb, bmT[...]) + bmb[...]
    lvb = _dot(ovb, bvT[...]) + bvb[...]
    ova = jnp.maximum(_dot(x, a1T[...]) + a1b[...], 0.0)
    ova = jnp.maximum(_dot(ova, a2T[...]) + a2b[...], 0.0)
    mua = _dot(ova, amT[...]) + amb[...]
    lva = _dot(ova, avT[...]) + avb[...]
    mu_r[...] = jnp.concatenate([mub, mua], axis=1)
    lv_r[...] = jnp.concatenate([lvb, lva], axis=1)


def _heads_enc(x, params):
    def tb(layers):
        out = []
        for wgt, b in layers:
            out += [wgt.T, b.reshape(1, -1)]
        return out
    ins = ([x] + tb(params['box_mean_var']) + tb(params['box_mean'])
           + tb(params['box_var']) + tb(params['angle_mean_var'])
           + tb(params['angle_mean']) + tb(params['angle_var']))
    specs = [_espec(256)] + [_wspec(a) for a in ins[1:]]
    out_shape = [jax.ShapeDtypeStruct((O_N, 128), F32)] * 2
    out_specs = [_espec(128)] * 2
    return pl.pallas_call(_heads_enc_body, grid=(GO,), in_specs=specs,
                          out_specs=out_specs, out_shape=out_shape)(*ins)


def _heads_dec_body(xd_r, z_r, at_r, b1T, b1b, b2T, b2b, n1T, n1b, n2T, n2b,
                    box_r, ang_r):
    xd = xd_r[...]
    z = z_r[...]
    xb = jnp.concatenate([xd, z, at_r[...]], axis=1)
    hb = jnp.maximum(_dot(xb, b1T[...]) + b1b[...], 0.0)
    box_r[...] = _dot(hb, b2T[...]) + b2b[...]
    xa = jnp.concatenate([xd, z], axis=1)
    ha = jnp.maximum(_dot(xa, n1T[...]) + n1b[...], 0.0)
    aa = _dot(ha, n2T[...]) + n2b[...]
    m = jnp.max(aa, axis=1, keepdims=True)
    ex = jnp.exp(aa - m)
    ang_r[...] = (aa - m) - jnp.log(jnp.sum(ex, axis=1, keepdims=True))


def _heads_dec(xd, z, attrd, params):
    def tb(layers):
        out = []
        for wgt, b in layers:
            out += [wgt.T, b.reshape(1, -1)]
        return out
    ins = [xd, z, attrd] + tb(params['box_net']) + tb(params['angle_net'])
    specs = [_espec(128), _espec(128), _espec(32)] + [_wspec(a) for a in ins[3:]]
    out_shape = [jax.ShapeDtypeStruct((O_N, 6), F32),
                 jax.ShapeDtypeStruct((O_N, NANG), F32)]
    out_specs = [pl.BlockSpec((TT, 6), lambda i: (i, 0)),
                 pl.BlockSpec((TT, NANG), lambda i: (i, 0))]
    return pl.pallas_call(_heads_dec_body, grid=(GO,), in_specs=specs,
                          out_specs=out_specs, out_shape=out_shape)(*ins)


@functools.lru_cache(None)
def _gather_builder(d):
    mesh = plsc.VectorSubcoreMesh(core_axis_name="c", subcore_axis_name="s")

    @functools.partial(
        pl.kernel,
        out_type=(jax.ShapeDtypeStruct((T_N, d), F32),
                  jax.ShapeDtypeStruct((T_N, d), F32)),
        mesh=mesh,
        scratch_types=[pltpu.VMEM((NCH, CB), I32), pltpu.VMEM((NCH, CB), I32),
                       pltpu.VMEM((CB, d), F32), pltpu.VMEM((CB, d), F32),
                       pltpu.SemaphoreType.DMA, pltpu.SemaphoreType.DMA])
    def gk(table, sidx, oidx, gs_out, go_out, sb, ob, ra, rb, sema, semb):
        cid = lax.axis_index("c")
        sid = lax.axis_index("s")
        wid = sid * NC + cid
        base = wid * PER_W
        pltpu.sync_copy(sidx.at[wid], sb)
        pltpu.sync_copy(oidx.at[wid], ob)

        def phase(ib, out):
            pltpu.async_copy(table.at[ib.at[0]], ra, sema)

            def body2(jj, carry):
                j = jj * 2
                pltpu.async_copy(table.at[ib.at[j + 1]], rb, semb)
                pltpu.make_async_copy(table.at[ib.at[0]], ra, sema).wait()
                pltpu.sync_copy(ra, out.at[pl.ds(base + j * CB, CB)])

                @pl.when(j + 2 < NCH)
                def _():
                    pltpu.async_copy(table.at[ib.at[j + 2]], ra, sema)

                pltpu.make_async_copy(table.at[ib.at[0]], rb, semb).wait()
                pltpu.sync_copy(rb, out.at[pl.ds(base + (j + 1) * CB, CB)])
                return carry

            lax.fori_loop(0, NCH // 2, body2, 0)
            if NCH % 2 == 1:
                pltpu.make_async_copy(table.at[ib.at[0]], ra, sema).wait()
                pltpu.sync_copy(ra, out.at[pl.ds(base + (NCH - 1) * CB, CB)])

        phase(sb, gs_out)
        phase(ob, go_out)

    return gk


def _gather_sc(table, sidx3, oidx3, d):
    return _gather_builder(d)(table, sidx3, oidx3)


@functools.lru_cache(None)
def _scatter_builder():
    mesh = plsc.VectorSubcoreMesh(core_axis_name="c", subcore_axis_name="s")

    @functools.partial(
        pl.kernel,
        out_type=jax.ShapeDtypeStruct((NC, 4, O_PAD, 128), F32),
        mesh=mesh,
        scratch_types=[pltpu.VMEM((NCH, CB), I32), pltpu.VMEM((NCH, CB), I32),
                       pltpu.VMEM((CB, 128), F32), pltpu.VMEM((CB, 128), F32),
                       pltpu.VMEM_SHARED((O_PAD, 128), F32)])
    def sk(s0, s1, s2, s3, o0, o1, o2, o3, sidx, oidx, zeros, out,
           sb, ob, va, vb, acc):
        cid = lax.axis_index("c")
        sid = lax.axis_index("s")
        wid = sid * NC + cid
        base = wid * PER_W
        pltpu.sync_copy(sidx.at[wid], sb)
        pltpu.sync_copy(oidx.at[wid], ob)
        svals = (s0, s1, s2, s3)
        ovals = (o0, o1, o2, o3)
        for c4 in range(4):
            pltpu.sync_copy(zeros.at[pl.ds(sid * STR, STR)],
                            acc.at[pl.ds(sid * STR, STR)])
            plsc.subcore_barrier()
            sv = svals[c4]
            ov = ovals[c4]

            def body(j, carry):
                pltpu.sync_copy(sv.at[pl.ds(base + j * CB, CB)], va)
                pltpu.sync_copy(va, acc.at[sb.at[j]], add=True)
                pltpu.sync_copy(ov.at[pl.ds(base + j * CB, CB)], vb)
                pltpu.sync_copy(vb, acc.at[ob.at[j]], add=True)
                return carry

            lax.fori_loop(0, NCH, body, 0)
            plsc.subcore_barrier()
            pltpu.sync_copy(acc.at[pl.ds(sid * STR, STR)],
                            out.at[cid, c4, pl.ds(sid * STR, STR)])

    return sk


def _scatter_sc(so_chunks, sidx3, oidx3, zeros128):
    return _scatter_builder()(*so_chunks, sidx3, oidx3, zeros128)


@functools.lru_cache(None)
def _counts_builder():
    mesh = plsc.VectorSubcoreMesh(core_axis_name="c", subcore_axis_name="s")

    @functools.partial(
        pl.kernel,
        out_type=jax.ShapeDtypeStruct((NC, O_PAD, 128), F32),
        mesh=mesh,
        scratch_types=[pltpu.VMEM((NCH, CB), I32), pltpu.VMEM((NCH, CB), I32),
                       pltpu.VMEM((CB, 128), F32),
                       pltpu.VMEM_SHARED((O_PAD, 128), F32)])
    def ck(sidx, oidx, ones, zeros, out, sb, ob, ov, acc):
        cid = lax.axis_index("c")
        sid = lax.axis_index("s")
        wid = sid * NC + cid
        pltpu.sync_copy(sidx.at[wid], sb)
        pltpu.sync_copy(oidx.at[wid], ob)
        pltpu.sync_copy(ones, ov)
        pltpu.sync_copy(zeros.at[pl.ds(sid * STR, STR)],
                        acc.at[pl.ds(sid * STR, STR)])
        plsc.subcore_barrier()

        def body(j, carry):
            pltpu.sync_copy(ov, acc.at[sb.at[j]], add=True)
            pltpu.sync_copy(ov, acc.at[ob.at[j]], add=True)
            return carry

        lax.fori_loop(0, NCH, body, 0)
        plsc.subcore_barrier()
        pltpu.sync_copy(acc.at[pl.ds(sid * STR, STR)],
                        out.at[cid, pl.ds(sid * STR, STR)])

    return ck


def _counts_sc(sidx3, oidx3, ones128, zeros128):
    return _counts_builder()(sidx3, oidx3, ones128, zeros128)


def _prep_gconv(p, din):
    (w1, b1), (w2, b2) = p['net1']
    w1T = w1.T
    w2T = w2.T
    (a1, ab1), (a2, ab2) = p['net2']
    return dict(
        a1sT=w1T[:din], a1pT=w1T[din:2 * din], a1oT=w1T[2 * din:],
        b1=b1.reshape(1, HID),
        w2sT=w2T[:, :HID], b2s=b2[:HID].reshape(1, HID),
        w2pT=w2T[:, HID:HID + din], b2p=b2[HID:HID + din].reshape(1, din),
        w2oT=w2T[:, HID + din:], b2o=b2[HID + din:].reshape(1, HID),
        n2aT=a1.T, n2b1=ab1.reshape(1, HID),
        n2bT=a2.T, n2b2=ab2.reshape(1, din),
    )


def kernel(objs, triples, boxes_gt, angles_gt, attributes, obj_to_img, params):
    objs = objs.astype(I32)
    attributes = attributes.astype(I32)
    angles_gt = angles_gt.astype(I32)
    s_idx = triples[:, 0].astype(I32)
    p_idx = triples[:, 1].astype(I32)
    o_idx = triples[:, 2].astype(I32)
    sidx3 = s_idx.reshape(NW, NCH, CB)
    oidx3 = o_idx.reshape(NW, NCH, CB)
    pidx3 = p_idx.reshape(GT, 1, TT)
    objs3 = objs.reshape(GO, 1, TT)
    attrs3 = attributes.reshape(GO, 1, TT)
    angs3 = angles_gt.reshape(GO, 1, TT)
    zeros128 = jnp.zeros((O_PAD, 128), F32)
    ones128 = jnp.ones((CB, 128), F32)

    we = [_prep_gconv(p, 256) for p in params['gconv_ec']]
    wd = [_prep_gconv(p, 128) for p in params['gconv_dc']]

    ove, ovd, attrd, pre1e, pre1d = _prep_call(
        objs3, attrs3, angs3, boxes_gt, params, we[0]['a1pT'], wd[0]['a1pT'])
    we[0]['pre1'] = pre1e
    wd[0]['pre1'] = pre1d

    counts = _counts_sc(sidx3, oidx3, ones128, zeros128)

    x = ove
    pv = pidx3
    for l in range(5):
        gs, go = _gather_sc(x, sidx3, oidx3, 256)
        outs = _edge_layer(gs, go, pv, we[l], l == 0, l == 4, 256)
        parts = _scatter_sc(tuple(outs[:8]), sidx3, oidx3, zeros128)
        x = _net2_layer(parts, counts, we[l], 256)
        pv = outs[8] if l < 4 else None

    mu, logvar = _heads_enc(x, params)

    xd = ovd
    pv = pidx3
    for l in range(5):
        gs, go = _gather_sc(xd, sidx3, oidx3, 128)
        outs = _edge_layer(gs, go, pv, wd[l], l == 0, l == 4, 128)
        parts = _scatter_sc(tuple(outs[:8]), sidx3, oidx3, zeros128)
        xd = _net2_layer(parts, counts, wd[l], 128)
        pv = outs[8] if l < 4 else None

    boxes_pred, angles_pred = _heads_dec(xd, mu, attrd, params)
    return (mu, logvar, boxes_pred, angles_pred)


# speedup vs baseline: 2.7937x; 1.2795x over previous
import functools

import jax
import jax.numpy as jnp
from jax import lax
from jax.experimental import pallas as pl
from jax.experimental.pallas import tpu as pltpu
from jax.experimental.pallas import tpu_sc as plsc

F32 = jnp.float32
I32 = jnp.int32

O_N = 10000
T_N = 160000
NO = 36
NP = 16
NA = 8
NANG = 24
HID = 512

NC, NS = 2, 16
NW = NC * NS
PER_W = T_N // NW
CB = 125
NCH = PER_W // CB
TB = T_N // CB
O_PAD = 10240
STR = O_PAD // NS

TT = 2000
GT = T_N // TT
GO = O_N // TT
EB = TT // CB


def _wspec(a):
    return pl.BlockSpec(a.shape, lambda i: (0,) * a.ndim)


def _espec(d):
    return pl.BlockSpec((TT, d), lambda i: (i, 0))


def _idxspec():
    return pl.BlockSpec((1, 1, TT), lambda i: (i, 0, 0))


def _onehot(v, n):
    return (v[:, None] == lax.broadcasted_iota(I32, (TT, n), 1)).astype(F32)


def _dot(a, b):
    return jnp.dot(a, b, preferred_element_type=F32)


def _prep_body(objs_r, attrs_r, angs_r, boxes_r, oec, aec, ange, wbT, bb,
               odc, adc, pec, a1pTe, pdc, a1pTd,
               ove_r, ovd_r, attrd_r, pre1e_r, pre1d_r):
    oh_obj = _onehot(objs_r[0, 0], NO)
    oh_att = _onehot(attrs_r[0, 0], NA)
    e_obj = _dot(oh_obj, oec[...])
    e_att = _dot(oh_att, aec[...])
    e_box = _dot(boxes_r[...], wbT[...]) + bb[...]
    e_ang = _dot(_onehot(angs_r[0, 0], NANG), ange[...])
    ove_r[...] = jnp.concatenate([e_obj, e_att, e_box, e_ang], axis=1)
    d_obj = _dot(oh_obj, odc[...])
    d_att = _dot(oh_att, adc[...])
    ovd_r[...] = jnp.concatenate([d_obj, d_att], axis=1)
    attrd_r[...] = d_att
    pre1e_r[...] = _dot(pec[...], a1pTe[...])
    pre1d_r[...] = _dot(pdc[...], a1pTd[...])


def _prep_call(objs3, attrs3, angs3, boxes, params, a1pTe, a1pTd):
    wb, bb = params['box_embeddings']
    ins = [objs3, attrs3, angs3, boxes,
           params['obj_emb_ec'], params['attr_emb_ec'], params['angle_emb'],
           wb.T, bb.reshape(1, -1),
           params['obj_emb_dc'], params['attr_emb_dc'],
           params['pred_emb_ec'], a1pTe, params['pred_emb_dc'], a1pTd]
    specs = [_idxspec(), _idxspec(), _idxspec(),
             pl.BlockSpec((TT, 6), lambda i: (i, 0))] + [_wspec(x) for x in ins[4:]]
    out_shape = [jax.ShapeDtypeStruct((O_N, 256), F32),
                 jax.ShapeDtypeStruct((O_N, 128), F32),
                 jax.ShapeDtypeStruct((O_N, 32), F32),
                 jax.ShapeDtypeStruct((NP, HID), F32),
                 jax.ShapeDtypeStruct((NP, HID), F32)]
    out_specs = [_espec(256), _espec(128), _espec(32),
                 pl.BlockSpec((NP, HID), lambda i: (0, 0)),
                 pl.BlockSpec((NP, HID), lambda i: (0, 0))]
    return pl.pallas_call(_prep_body, grid=(GO,), in_specs=specs,
                          out_specs=out_specs, out_shape=out_shape)(*ins)


def _edge_body(layer1, last):
    def body(*refs):
        if layer1:
            pv, gs, go, pmat, a1sT, a1oT, b1, w2sT, b2s, w2oT, b2o, *rest = refs
        else:
            pv, gs, go, pmat, a1sT, a1oT, b1, w2sT, b2s, w2oT, b2o, *rest = refs
        if not last:
            w2pT, b2p, *rest = rest
        outs = rest
        gsm = gs[...].reshape(TT, gs.shape[2])
        gom = go[...].reshape(TT, go.shape[2])
        h = _dot(gsm, a1sT[...]) + _dot(gom, a1oT[...])
        if layer1:
            h += _dot(_onehot(pv[0, 0], NP), pmat[...])
        else:
            h += _dot(pv[...], pmat[...])
        h = jnp.maximum(h + b1[...], 0.0)
        s = jnp.maximum(_dot(h, w2sT[...]) + b2s[...], 0.0)
        o = jnp.maximum(_dot(h, w2oT[...]) + b2o[...], 0.0)
        for k in range(4):
            outs[k][...] = s[:, k * 128:(k + 1) * 128].reshape(EB, CB, 128)
            outs[4 + k][...] = o[:, k * 128:(k + 1) * 128].reshape(EB, CB, 128)
        if not last:
            outs[8][...] = jnp.maximum(_dot(h, w2pT[...]) + b2p[...], 0.0)
    return body


def _gspec(d):
    return pl.BlockSpec((EB, CB, d), lambda i: (i, 0, 0))


def _edge_layer(gs, go, pvp, w, layer1, last, din):
    if layer1:
        ins = [pvp, gs, go, w['pre1']]
        specs = [_idxspec(), _gspec(din), _gspec(din), _wspec(w['pre1'])]
    else:
        ins = [pvp, gs, go, w['a1pT']]
        specs = [_espec(din), _gspec(din), _gspec(din), _wspec(w['a1pT'])]
    tail = [w['a1sT'], w['a1oT'], w['b1'], w['w2sT'], w['b2s'], w['w2oT'], w['b2o']]
    if not last:
        tail += [w['w2pT'], w['b2p']]
    ins += tail
    specs += [_wspec(x) for x in tail]
    out_shape = [jax.ShapeDtypeStruct((TB, CB, 128), F32) for _ in range(8)]
    out_specs = [_gspec(128) for _ in range(8)]
    if not last:
        out_shape.append(jax.ShapeDtypeStruct((T_N, din), F32))
        out_specs.append(_espec(din))
    return pl.pallas_call(_edge_body(layer1, last), grid=(GT,), in_specs=specs,
                          out_specs=out_specs, out_shape=out_shape)(*ins)


def _net2_body(p_r, c_r, aT, ab, bT, bb, out_r):
    cnt = c_r[0, :, 0:1] + c_r[1, :, 0:1]
    inv = 1.0 / jnp.maximum(cnt, 1.0)
    aTm = aT[...]
    h = jnp.zeros((TT, HID), F32) + ab[...]
    for k in range(4):
        pk = (p_r[0, k] + p_r[1, k]) * inv
        h += _dot(pk, aTm[k * 128:(k + 1) * 128])
    h = jnp.maximum(h, 0.0)
    out_r[...] = jnp.maximum(_dot(h, bT[...]) + bb[...], 0.0)


def _net2_layer(parts, counts, w, dn):
    ins = [parts, counts, w['n2aT'], w['n2b1'], w['n2bT'], w['n2b2']]
    specs = [pl.BlockSpec((NC, 4, TT, 128), lambda i: (0, 0, i, 0)),
             pl.BlockSpec((NC, TT, 128), lambda i: (0, i, 0)),
             _wspec(w['n2aT']), _wspec(w['n2b1']),
             _wspec(w['n2bT']), _wspec(w['n2b2'])]
    return pl.pallas_call(_net2_body, grid=(GO,), in_specs=specs,
                          out_specs=_espec(dn),
                          out_shape=jax.ShapeDtypeStruct((O_N, dn), F32))(*ins)


def _heads_enc_body(x_r, m1T, m1b, m2T, m2b, bmT, bmb, bvT, bvb,
                    a1T, a1b, a2T, a2b, amT, amb, avT, avb, mu_r, lv_r):
    x = x_r[...]
    ovb = jnp.maximum(_dot(x, m1T[...]) + m1b[...], 0.0)
    ovb = jnp.maximum(_dot(ovb, m2T[...]) + m2b[...], 0.0)
    mub = _dot(ovb, bmT[...]) + bmb[...]
    lvb = _dot(ovb, bvT[...]) + bvb[...]
    ova = jnp.maximum(_dot(x, a1T[...]) + a1b[...], 0.0)
    ova = jnp.maximum(_dot(ova, a2T[...]) + a2b[...], 0.0)
    mua = _dot(ova, amT[...]) + amb[...]
    lva = _dot(ova, avT[...]) + avb[...]
    mu_r[...] = jnp.concatenate([mub, mua], axis=1)
    lv_r[...] = jnp.concatenate([lvb, lva], axis=1)


def _heads_enc(x, params):
    def tb(layers):
        out = []
        for wgt, b in layers:
            out += [wgt.T, b.reshape(1, -1)]
        return out
    ins = ([x] + tb(params['box_mean_var']) + tb(params['box_mean'])
           + tb(params['box_var']) + tb(params['angle_mean_var'])
           + tb(params['angle_mean']) + tb(params['angle_var']))
    specs = [_espec(256)] + [_wspec(a) for a in ins[1:]]
    out_shape = [jax.ShapeDtypeStruct((O_N, 128), F32)] * 2
    out_specs = [_espec(128)] * 2
    return pl.pallas_call(_heads_enc_body, grid=(GO,), in_specs=specs,
                          out_specs=out_specs, out_shape=out_shape)(*ins)


def _heads_dec_body(xd_r, z_r, at_r, b1T, b1b, b2T, b2b, n1T, n1b, n2T, n2b,
                    box_r, ang_r):
    xd = xd_r[...]
    z = z_r[...]
    xb = jnp.concatenate([xd, z, at_r[...]], axis=1)
    hb = jnp.maximum(_dot(xb, b1T[...]) + b1b[...], 0.0)
    box_r[...] = _dot(hb, b2T[...]) + b2b[...]
    xa = jnp.concatenate([xd, z], axis=1)
    ha = jnp.maximum(_dot(xa, n1T[...]) + n1b[...], 0.0)
    aa = _dot(ha, n2T[...]) + n2b[...]
    m = jnp.max(aa, axis=1, keepdims=True)
    ex = jnp.exp(aa - m)
    ang_r[...] = (aa - m) - jnp.log(jnp.sum(ex, axis=1, keepdims=True))


def _heads_dec(xd, z, attrd, params):
    def tb(layers):
        out = []
        for wgt, b in layers:
            out += [wgt.T, b.reshape(1, -1)]
        return out
    ins = [xd, z, attrd] + tb(params['box_net']) + tb(params['angle_net'])
    specs = [_espec(128), _espec(128), _espec(32)] + [_wspec(a) for a in ins[3:]]
    out_shape = [jax.ShapeDtypeStruct((O_N, 6), F32),
                 jax.ShapeDtypeStruct((O_N, NANG), F32)]
    out_specs = [pl.BlockSpec((TT, 6), lambda i: (i, 0)),
                 pl.BlockSpec((TT, NANG), lambda i: (i, 0))]
    return pl.pallas_call(_heads_dec_body, grid=(GO,), in_specs=specs,
                          out_specs=out_specs, out_shape=out_shape)(*ins)


@functools.lru_cache(None)
def _gather_builder(d):
    mesh = plsc.VectorSubcoreMesh(core_axis_name="c", subcore_axis_name="s")

    @functools.partial(
        pl.kernel,
        out_type=(jax.ShapeDtypeStruct((TB, CB, d), F32),
                  jax.ShapeDtypeStruct((TB, CB, d), F32)),
        mesh=mesh,
        scratch_types=[pltpu.VMEM((NCH, CB), I32), pltpu.VMEM((NCH, CB), I32),
                       pltpu.VMEM((1, CB, d), F32), pltpu.VMEM((1, CB, d), F32),
                       pltpu.SemaphoreType.DMA, pltpu.SemaphoreType.DMA])
    def gk(table, sidx, oidx, gs_out, go_out, sb, ob, ra, rb, sema, semb):
        cid = lax.axis_index("c")
        sid = lax.axis_index("s")
        wid = sid * NC + cid
        wblk = wid * NCH
        pltpu.sync_copy(sidx.at[wid], sb)
        pltpu.sync_copy(oidx.at[wid], ob)

        def phase(ib, out):
            pltpu.async_copy(table.at[ib.at[0]], ra.at[0], sema)

            def body2(cc, carry):
                c = cc * 2
                pltpu.async_copy(table.at[ib.at[c + 1]], rb.at[0], semb)
                pltpu.make_async_copy(table.at[ib.at[0]], ra.at[0], sema).wait()
                pltpu.sync_copy(ra, out.at[pl.ds(wblk + c, 1)])

                @pl.when(c + 2 < NCH)
                def _():
                    pltpu.async_copy(table.at[ib.at[c + 2]], ra.at[0], sema)

                pltpu.make_async_copy(table.at[ib.at[0]], rb.at[0], semb).wait()
                pltpu.sync_copy(rb, out.at[pl.ds(wblk + c + 1, 1)])
                return carry

            lax.fori_loop(0, NCH // 2, body2, 0)

        phase(sb, gs_out)
        phase(ob, go_out)

    return gk


def _gather_sc(table, sidx3, oidx3, d):
    return _gather_builder(d)(table, sidx3, oidx3)


@functools.lru_cache(None)
def _scatter_builder():
    mesh = plsc.VectorSubcoreMesh(core_axis_name="c", subcore_axis_name="s")

    @functools.partial(
        pl.kernel,
        out_type=jax.ShapeDtypeStruct((NC, 4, O_PAD, 128), F32),
        mesh=mesh,
        scratch_types=[pltpu.VMEM((NCH, CB), I32), pltpu.VMEM((NCH, CB), I32),
                       pltpu.VMEM((1, CB, 128), F32),
                       pltpu.VMEM((1, CB, 128), F32),
                       pltpu.VMEM_SHARED((O_PAD, 128), F32),
                       pltpu.SemaphoreType.DMA, pltpu.SemaphoreType.DMA])
    def sk(s0, s1, s2, s3, o0, o1, o2, o3, sidx, oidx, zeros, out,
           sb, ob, va, vb, acc, m1, m2):
        cid = lax.axis_index("c")
        sid = lax.axis_index("s")
        wid = sid * NC + cid
        wblk = wid * NCH
        pltpu.sync_copy(sidx.at[wid], sb)
        pltpu.sync_copy(oidx.at[wid], ob)
        svals = (s0, s1, s2, s3)
        ovals = (o0, o1, o2, o3)
        for c4 in range(4):
            pltpu.sync_copy(zeros.at[pl.ds(sid * STR, STR)],
                            acc.at[pl.ds(sid * STR, STR)])
            plsc.subcore_barrier()
            sv = svals[c4]
            ov = ovals[c4]
            pltpu.async_copy(sv.at[pl.ds(wblk, 1)], va, m1)
            pltpu.async_copy(ov.at[pl.ds(wblk, 1)], vb, m2)

            def body(j, carry):
                pltpu.make_async_copy(sv.at[pl.ds(wblk, 1)], va, m1).wait()
                pltpu.sync_copy(va.at[0], acc.at[sb.at[j]], add=True)

                @pl.when(j + 1 < NCH)
                def _():
                    pltpu.async_copy(sv.at[pl.ds(wblk + j + 1, 1)], va, m1)

                pltpu.make_async_copy(ov.at[pl.ds(wblk, 1)], vb, m2).wait()
                pltpu.sync_copy(vb.at[0], acc.at[ob.at[j]], add=True)

                @pl.when(j + 1 < NCH)
                def _():
                    pltpu.async_copy(ov.at[pl.ds(wblk + j + 1, 1)], vb, m2)

                return carry

            lax.fori_loop(0, NCH, body, 0)
            plsc.subcore_barrier()
            pltpu.sync_copy(acc.at[pl.ds(sid * STR, STR)],
                            out.at[cid, c4, pl.ds(sid * STR, STR)])

    return sk


def _scatter_sc(so_chunks, sidx3, oidx3, zeros128):
    return _scatter_builder()(*so_chunks, sidx3, oidx3, zeros128)


@functools.lru_cache(None)
def _counts_builder():
    mesh = plsc.VectorSubcoreMesh(core_axis_name="c", subcore_axis_name="s")

    @functools.partial(
        pl.kernel,
        out_type=jax.ShapeDtypeStruct((NC, O_PAD, 128), F32),
        mesh=mesh,
        scratch_types=[pltpu.VMEM((NCH, CB), I32), pltpu.VMEM((NCH, CB), I32),
                       pltpu.VMEM((CB, 128), F32),
                       pltpu.VMEM_SHARED((O_PAD, 128), F32)])
    def ck(sidx, oidx, ones, zeros, out, sb, ob, ov, acc):
        cid = lax.axis_index("c")
        sid = lax.axis_index("s")
        wid = sid * NC + cid
        pltpu.sync_copy(sidx.at[wid], sb)
        pltpu.sync_copy(oidx.at[wid], ob)
        pltpu.sync_copy(ones, ov)
        pltpu.sync_copy(zeros.at[pl.ds(sid * STR, STR)],
                        acc.at[pl.ds(sid * STR, STR)])
        plsc.subcore_barrier()

        def body(j, carry):
            pltpu.sync_copy(ov, acc.at[sb.at[j]], add=True)
            pltpu.sync_copy(ov, acc.at[ob.at[j]], add=True)
            return carry

        lax.fori_loop(0, NCH, body, 0)
        plsc.subcore_barrier()
        pltpu.sync_copy(acc.at[pl.ds(sid * STR, STR)],
                        out.at[cid, pl.ds(sid * STR, STR)])

    return ck


def _counts_sc(sidx3, oidx3, ones128, zeros128):
    return _counts_builder()(sidx3, oidx3, ones128, zeros128)


def _prep_gconv(p, din):
    (w1, b1), (w2, b2) = p['net1']
    w1T = w1.T
    w2T = w2.T
    (a1, ab1), (a2, ab2) = p['net2']
    return dict(
        a1sT=w1T[:din], a1pT=w1T[din:2 * din], a1oT=w1T[2 * din:],
        b1=b1.reshape(1, HID),
        w2sT=w2T[:, :HID], b2s=b2[:HID].reshape(1, HID),
        w2pT=w2T[:, HID:HID + din], b2p=b2[HID:HID + din].reshape(1, din),
        w2oT=w2T[:, HID + din:], b2o=b2[HID + din:].reshape(1, HID),
        n2aT=a1.T, n2b1=ab1.reshape(1, HID),
        n2bT=a2.T, n2b2=ab2.reshape(1, din),
    )


def kernel(objs, triples, boxes_gt, angles_gt, attributes, obj_to_img, params):
    objs = objs.astype(I32)
    attributes = attributes.astype(I32)
    angles_gt = angles_gt.astype(I32)
    s_idx = triples[:, 0].astype(I32)
    p_idx = triples[:, 1].astype(I32)
    o_idx = triples[:, 2].astype(I32)
    sidx3 = s_idx.reshape(NW, NCH, CB)
    oidx3 = o_idx.reshape(NW, NCH, CB)
    pidx3 = p_idx.reshape(GT, 1, TT)
    objs3 = objs.reshape(GO, 1, TT)
    attrs3 = attributes.reshape(GO, 1, TT)
    angs3 = angles_gt.reshape(GO, 1, TT)
    zeros128 = jnp.zeros((O_PAD, 128), F32)
    ones128 = jnp.ones((CB, 128), F32)

    we = [_prep_gconv(p, 256) for p in params['gconv_ec']]
    wd = [_prep_gconv(p, 128) for p in params['gconv_dc']]

    ove, ovd, attrd, pre1e, pre1d = _prep_call(
        objs3, attrs3, angs3, boxes_gt, params, we[0]['a1pT'], wd[0]['a1pT'])
    we[0]['pre1'] = pre1e
    wd[0]['pre1'] = pre1d

    counts = _counts_sc(sidx3, oidx3, ones128, zeros128)

    x = ove
    pv = pidx3
    for l in range(5):
        gs, go = _gather_sc(x, sidx3, oidx3, 256)
        outs = _edge_layer(gs, go, pv, we[l], l == 0, l == 4, 256)
        parts = _scatter_sc(tuple(outs[:8]), sidx3, oidx3, zeros128)
        x = _net2_layer(parts, counts, we[l], 256)
        pv = outs[8] if l < 4 else None

    mu, logvar = _heads_enc(x, params)

    xd = ovd
    pv = pidx3
    for l in range(5):
        gs, go = _gather_sc(xd, sidx3, oidx3, 128)
        outs = _edge_layer(gs, go, pv, wd[l], l == 0, l == 4, 128)
        parts = _scatter_sc(tuple(outs[:8]), sidx3, oidx3, zeros128)
        xd = _net2_layer(parts, counts, wd[l], 128)
        pv = outs[8] if l < 4 else None

    boxes_pred, angles_pred = _heads_dec(xd, mu, attrd, params)
    return (mu, logvar, boxes_pred, angles_pred)


# speedup vs baseline: 2.9610x; 1.0599x over previous
import functools

import jax
import jax.numpy as jnp
from jax import lax
from jax.experimental import pallas as pl
from jax.experimental.pallas import tpu as pltpu
from jax.experimental.pallas import tpu_sc as plsc

F32 = jnp.float32
BF16 = jnp.bfloat16
I32 = jnp.int32

O_N = 10000
T_N = 160000
NO = 36
NP = 16
NA = 8
NANG = 24
HID = 512

NC, NS = 2, 16
NW = NC * NS
PER_W = T_N // NW
CB = 125
NCH = PER_W // CB
TB = T_N // CB
O_PAD = 10240
STR = O_PAD // NS

TT = 2000
GT = T_N // TT
GO = O_N // TT
EB = TT // CB


def _wspec(a):
    return pl.BlockSpec(a.shape, lambda i: (0,) * a.ndim)


def _espec(d):
    return pl.BlockSpec((TT, d), lambda i: (i, 0))


def _idxspec():
    return pl.BlockSpec((1, 1, TT), lambda i: (i, 0, 0))


def _onehot(v, n, dt=F32):
    return (v[:, None] == lax.broadcasted_iota(I32, (TT, n), 1)).astype(dt)


def _dot(a, b):
    return jnp.dot(a, b, preferred_element_type=F32)


def _rne16(u):
    return (u + jnp.uint32(0x7FFF) + ((u >> 16) & jnp.uint32(1))) >> 16


def _pack(x):
    d2 = x.shape[1] // 2
    lo = _rne16(lax.bitcast_convert_type(x[:, :d2], jnp.uint32))
    hi = _rne16(lax.bitcast_convert_type(x[:, d2:], jnp.uint32))
    return lax.bitcast_convert_type(lo | (hi << 16), F32)


def _unpack(g):
    w = lax.bitcast_convert_type(g, jnp.uint32)
    lo = lax.bitcast_convert_type(w << 16, F32)
    hi = lax.bitcast_convert_type(w & jnp.uint32(0xFFFF0000), F32)
    return jnp.concatenate([lo, hi], axis=1).astype(BF16)


def _prep_body(objs_r, attrs_r, angs_r, boxes_r, oec, aec, ange, wbT, bb,
               odc, adc, pec, a1pTe, pdc, a1pTd,
               ove_r, ovd_r, attrd_r, pre1e_r, pre1d_r):
    oh_obj = _onehot(objs_r[0, 0], NO)
    oh_att = _onehot(attrs_r[0, 0], NA)
    e_obj = _dot(oh_obj, oec[...])
    e_att = _dot(oh_att, aec[...])
    e_box = _dot(boxes_r[...], wbT[...]) + bb[...]
    e_ang = _dot(_onehot(angs_r[0, 0], NANG), ange[...])
    ove_r[...] = _pack(jnp.concatenate([e_obj, e_att, e_box, e_ang], axis=1))
    d_obj = _dot(oh_obj, odc[...])
    d_att = _dot(oh_att, adc[...])
    ovd_r[...] = jnp.concatenate([d_obj, d_att], axis=1)
    attrd_r[...] = d_att
    pre1e_r[...] = _dot(pec[...], a1pTe[...]).astype(BF16)
    pre1d_r[...] = _dot(pdc[...], a1pTd[...]).astype(BF16)


def _prep_call(objs3, attrs3, angs3, boxes, params, a1pTe, a1pTd):
    wb, bb = params['box_embeddings']
    ins = [objs3, attrs3, angs3, boxes,
           params['obj_emb_ec'], params['attr_emb_ec'], params['angle_emb'],
           wb.T, bb.reshape(1, -1),
           params['obj_emb_dc'], params['attr_emb_dc'],
           params['pred_emb_ec'], a1pTe, params['pred_emb_dc'], a1pTd]
    specs = [_idxspec(), _idxspec(), _idxspec(),
             pl.BlockSpec((TT, 6), lambda i: (i, 0))] + [_wspec(x) for x in ins[4:]]
    out_shape = [jax.ShapeDtypeStruct((O_N, 128), F32),
                 jax.ShapeDtypeStruct((O_N, 128), F32),
                 jax.ShapeDtypeStruct((O_N, 32), F32),
                 jax.ShapeDtypeStruct((NP, HID), BF16),
                 jax.ShapeDtypeStruct((NP, HID), BF16)]
    out_specs = [_espec(128), _espec(128), _espec(32),
                 pl.BlockSpec((NP, HID), lambda i: (0, 0)),
                 pl.BlockSpec((NP, HID), lambda i: (0, 0))]
    return pl.pallas_call(_prep_body, grid=(GO,), in_specs=specs,
                          out_specs=out_specs, out_shape=out_shape)(*ins)


def _edge_body(layer1, last, packed):
    def body(*refs):
        if layer1:
            pv, gs, go, pmat, a1sT, a1oT, b1, w2sT, b2s, w2oT, b2o, *rest = refs
        else:
            pv, gs, go, pmat, a1sT, a1oT, b1, w2sT, b2s, w2oT, b2o, *rest = refs
        if not last:
            w2pT, b2p, *rest = rest
        outs = rest
        if packed:
            gsm = _unpack(gs[...].reshape(TT, gs.shape[2]))
            gom = _unpack(go[...].reshape(TT, go.shape[2]))
        else:
            gsm = gs[...].reshape(TT, gs.shape[2]).astype(BF16)
            gom = go[...].reshape(TT, go.shape[2]).astype(BF16)
        h = _dot(gsm, a1sT[...]) + _dot(gom, a1oT[...])
        if layer1:
            h += _dot(_onehot(pv[0, 0], NP, BF16), pmat[...])
        else:
            h += _dot(pv[...], pmat[...])
        hb = jnp.maximum(h + b1[...], 0.0).astype(BF16)
        s = jnp.maximum(_dot(hb, w2sT[...]) + b2s[...], 0.0)
        o = jnp.maximum(_dot(hb, w2oT[...]) + b2o[...], 0.0)
        for k in range(4):
            outs[k][...] = s[:, k * 128:(k + 1) * 128].reshape(EB, CB, 128)
            outs[4 + k][...] = o[:, k * 128:(k + 1) * 128].reshape(EB, CB, 128)
        if not last:
            outs[8][...] = jnp.maximum(
                _dot(hb, w2pT[...]) + b2p[...], 0.0).astype(BF16)
    return body


def _gspec(d):
    return pl.BlockSpec((EB, CB, d), lambda i: (i, 0, 0))


def _edge_layer(gs, go, pvp, w, layer1, last, din, packed):
    d2 = din // 2 if packed else din
    if layer1:
        ins = [pvp, gs, go, w['pre1']]
        specs = [_idxspec(), _gspec(d2), _gspec(d2), _wspec(w['pre1'])]
    else:
        ins = [pvp, gs, go, w['a1pT']]
        specs = [_espec(din), _gspec(d2), _gspec(d2), _wspec(w['a1pT'])]
    tail = [w['a1sT'], w['a1oT'], w['b1'], w['w2sT'], w['b2s'], w['w2oT'], w['b2o']]
    if not last:
        tail += [w['w2pT'], w['b2p']]
    ins += tail
    specs += [_wspec(x) for x in tail]
    out_shape = [jax.ShapeDtypeStruct((TB, CB, 128), F32) for _ in range(8)]
    out_specs = [_gspec(128) for _ in range(8)]
    if not last:
        out_shape.append(jax.ShapeDtypeStruct((T_N, din), BF16))
        out_specs.append(_espec(din))
    return pl.pallas_call(_edge_body(layer1, last, packed), grid=(GT,), in_specs=specs,
                          out_specs=out_specs, out_shape=out_shape)(*ins)


def _net2_body(p_r, c_r, aT, ab, bT, bb, out_r):
    cnt = c_r[0, :, 0:1] + c_r[1, :, 0:1]
    inv = 1.0 / jnp.maximum(cnt, 1.0)
    aTm = aT[...]
    h = jnp.zeros((TT, HID), F32) + ab[...]
    for k in range(4):
        pk = (p_r[0, k] + p_r[1, k]) * inv
        h += _dot(pk, aTm[k * 128:(k + 1) * 128])
    h = jnp.maximum(h, 0.0)
    res = jnp.maximum(_dot(h, bT[...]) + bb[...], 0.0)
    if out_r.shape[1] * 2 == bT.shape[1]:
        out_r[...] = _pack(res)
    else:
        out_r[...] = res


def _net2_layer(parts, counts, w, dn, last=False):
    ins = [parts, counts, w['n2aT'], w['n2b1'], w['n2bT'], w['n2b2']]
    specs = [pl.BlockSpec((NC, 4, TT, 128), lambda i: (0, 0, i, 0)),
             pl.BlockSpec((NC, TT, 128), lambda i: (0, i, 0)),
             _wspec(w['n2aT']), _wspec(w['n2b1']),
             _wspec(w['n2bT']), _wspec(w['n2b2'])]
    dn_out = dn // 2 if (not last and dn == 256) else dn
    return pl.pallas_call(_net2_body, grid=(GO,), in_specs=specs,
                          out_specs=_espec(dn_out),
                          out_shape=jax.ShapeDtypeStruct((O_N, dn_out), F32))(*ins)


def _heads_enc_body(x_r, m1T, m1b, m2T, m2b, bmT, bmb, bvT, bvb,
                    a1T, a1b, a2T, a2b, amT, amb, avT, avb, mu_r, lv_r):
    x = x_r[...]
    ovb = jnp.maximum(_dot(x, m1T[...]) + m1b[...], 0.0)
    ovb = jnp.maximum(_dot(ovb, m2T[...]) + m2b[...], 0.0)
    mub = _dot(ovb, bmT[...]) + bmb[...]
    lvb = _dot(ovb, bvT[...]) + bvb[...]
    ova = jnp.maximum(_dot(x, a1T[...]) + a1b[...], 0.0)
    ova = jnp.maximum(_dot(ova, a2T[...]) + a2b[...], 0.0)
    mua = _dot(ova, amT[...]) + amb[...]
    lva = _dot(ova, avT[...]) + avb[...]
    mu_r[...] = jnp.concatenate([mub, mua], axis=1)
    lv_r[...] = jnp.concatenate([lvb, lva], axis=1)


def _heads_enc(x, params):
    def tb(layers):
        out = []
        for wgt, b in layers:
            out += [wgt.T, b.reshape(1, -1)]
        return out
    ins = ([x] + tb(params['box_mean_var']) + tb(params['box_mean'])
           + tb(params['box_var']) + tb(params['angle_mean_var'])
           + tb(params['angle_mean']) + tb(params['angle_var']))
    specs = [_espec(256)] + [_wspec(a) for a in ins[1:]]
    out_shape = [jax.ShapeDtypeStruct((O_N, 128), F32)] * 2
    out_specs = [_espec(128)] * 2
    return pl.pallas_call(_heads_enc_body, grid=(GO,), in_specs=specs,
                          out_specs=out_specs, out_shape=out_shape)(*ins)


def _heads_dec_body(xd_r, z_r, at_r, b1T, b1b, b2T, b2b, n1T, n1b, n2T, n2b,
                    box_r, ang_r):
    xd = xd_r[...]
    z = z_r[...]
    xb = jnp.concatenate([xd, z, at_r[...]], axis=1)
    hb = jnp.maximum(_dot(xb, b1T[...]) + b1b[...], 0.0)
    box_r[...] = _dot(hb, b2T[...]) + b2b[...]
    xa = jnp.concatenate([xd, z], axis=1)
    ha = jnp.maximum(_dot(xa, n1T[...]) + n1b[...], 0.0)
    aa = _dot(ha, n2T[...]) + n2b[...]
    m = jnp.max(aa, axis=1, keepdims=True)
    ex = jnp.exp(aa - m)
    ang_r[...] = (aa - m) - jnp.log(jnp.sum(ex, axis=1, keepdims=True))


def _heads_dec(xd, z, attrd, params):
    def tb(layers):
        out = []
        for wgt, b in layers:
            out += [wgt.T, b.reshape(1, -1)]
        return out
    ins = [xd, z, attrd] + tb(params['box_net']) + tb(params['angle_net'])
    specs = [_espec(128), _espec(128), _espec(32)] + [_wspec(a) for a in ins[3:]]
    out_shape = [jax.ShapeDtypeStruct((O_N, 6), F32),
                 jax.ShapeDtypeStruct((O_N, NANG), F32)]
    out_specs = [pl.BlockSpec((TT, 6), lambda i: (i, 0)),
                 pl.BlockSpec((TT, NANG), lambda i: (i, 0))]
    return pl.pallas_call(_heads_dec_body, grid=(GO,), in_specs=specs,
                          out_specs=out_specs, out_shape=out_shape)(*ins)


@functools.lru_cache(None)
def _gather_builder(d):
    mesh = plsc.VectorSubcoreMesh(core_axis_name="c", subcore_axis_name="s")

    @functools.partial(
        pl.kernel,
        out_type=(jax.ShapeDtypeStruct((TB, CB, d), F32),
                  jax.ShapeDtypeStruct((TB, CB, d), F32)),
        mesh=mesh,
        scratch_types=[pltpu.VMEM((NCH, CB), I32), pltpu.VMEM((NCH, CB), I32),
                       pltpu.VMEM((1, CB, d), F32), pltpu.VMEM((1, CB, d), F32),
                       pltpu.SemaphoreType.DMA, pltpu.SemaphoreType.DMA])
    def gk(table, sidx, oidx, gs_out, go_out, sb, ob, ra, rb, sema, semb):
        cid = lax.axis_index("c")
        sid = lax.axis_index("s")
        wid = sid * NC + cid
        wblk = wid * NCH
        pltpu.sync_copy(sidx.at[wid], sb)
        pltpu.sync_copy(oidx.at[wid], ob)

        def phase(ib, out):
            pltpu.async_copy(table.at[ib.at[0]], ra.at[0], sema)

            def body2(cc, carry):
                c = cc * 2
                pltpu.async_copy(table.at[ib.at[c + 1]], rb.at[0], semb)
                pltpu.make_async_copy(table.at[ib.at[0]], ra.at[0], sema).wait()
                pltpu.sync_copy(ra, out.at[pl.ds(wblk + c, 1)])

                @pl.when(c + 2 < NCH)
                def _():
                    pltpu.async_copy(table.at[ib.at[c + 2]], ra.at[0], sema)

                pltpu.make_async_copy(table.at[ib.at[0]], rb.at[0], semb).wait()
                pltpu.sync_copy(rb, out.at[pl.ds(wblk + c + 1, 1)])
                return carry

            lax.fori_loop(0, NCH // 2, body2, 0)

        phase(sb, gs_out)
        phase(ob, go_out)

    return gk


def _gather_sc(table, sidx3, oidx3, d):
    return _gather_builder(d)(table, sidx3, oidx3)


@functools.lru_cache(None)
def _scatter_builder():
    mesh = plsc.VectorSubcoreMesh(core_axis_name="c", subcore_axis_name="s")

    @functools.partial(
        pl.kernel,
        out_type=jax.ShapeDtypeStruct((NC, 4, O_PAD, 128), F32),
        mesh=mesh,
        scratch_types=[pltpu.VMEM((NCH, CB), I32), pltpu.VMEM((NCH, CB), I32),
                       pltpu.VMEM((1, CB, 128), F32),
                       pltpu.VMEM((1, CB, 128), F32),
                       pltpu.VMEM_SHARED((O_PAD, 128), F32),
                       pltpu.SemaphoreType.DMA, pltpu.SemaphoreType.DMA])
    def sk(s0, s1, s2, s3, o0, o1, o2, o3, sidx, oidx, zeros, out,
           sb, ob, va, vb, acc, m1, m2):
        cid = lax.axis_index("c")
        sid = lax.axis_index("s")
        wid = sid * NC + cid
        wblk = wid * NCH
        pltpu.sync_copy(sidx.at[wid], sb)
        pltpu.sync_copy(oidx.at[wid], ob)
        svals = (s0, s1, s2, s3)
        ovals = (o0, o1, o2, o3)
        for c4 in range(4):
            pltpu.sync_copy(zeros.at[pl.ds(sid * STR, STR)],
                            acc.at[pl.ds(sid * STR, STR)])
            plsc.subcore_barrier()
            sv = svals[c4]
            ov = ovals[c4]
            pltpu.async_copy(sv.at[pl.ds(wblk, 1)], va, m1)
            pltpu.async_copy(ov.at[pl.ds(wblk, 1)], vb, m2)

            def body(j, carry):
                pltpu.make_async_copy(sv.at[pl.ds(wblk, 1)], va, m1).wait()
                pltpu.sync_copy(va.at[0], acc.at[sb.at[j]], add=True)

                @pl.when(j + 1 < NCH)
                def _():
                    pltpu.async_copy(sv.at[pl.ds(wblk + j + 1, 1)], va, m1)

                pltpu.make_async_copy(ov.at[pl.ds(wblk, 1)], vb, m2).wait()
                pltpu.sync_copy(vb.at[0], acc.at[ob.at[j]], add=True)

                @pl.when(j + 1 < NCH)
                def _():
                    pltpu.async_copy(ov.at[pl.ds(wblk + j + 1, 1)], vb, m2)

                return carry

            lax.fori_loop(0, NCH, body, 0)
            plsc.subcore_barrier()
            pltpu.sync_copy(acc.at[pl.ds(sid * STR, STR)],
                            out.at[cid, c4, pl.ds(sid * STR, STR)])

    return sk


def _scatter_sc(so_chunks, sidx3, oidx3, zeros128):
    return _scatter_builder()(*so_chunks, sidx3, oidx3, zeros128)


@functools.lru_cache(None)
def _counts_builder():
    mesh = plsc.VectorSubcoreMesh(core_axis_name="c", subcore_axis_name="s")

    @functools.partial(
        pl.kernel,
        out_type=jax.ShapeDtypeStruct((NC, O_PAD, 128), F32),
        mesh=mesh,
        scratch_types=[pltpu.VMEM((NCH, CB), I32), pltpu.VMEM((NCH, CB), I32),
                       pltpu.VMEM((CB, 128), F32),
                       pltpu.VMEM_SHARED((O_PAD, 128), F32)])
    def ck(sidx, oidx, ones, zeros, out, sb, ob, ov, acc):
        cid = lax.axis_index("c")
        sid = lax.axis_index("s")
        wid = sid * NC + cid
        pltpu.sync_copy(sidx.at[wid], sb)
        pltpu.sync_copy(oidx.at[wid], ob)
        pltpu.sync_copy(ones, ov)
        pltpu.sync_copy(zeros.at[pl.ds(sid * STR, STR)],
                        acc.at[pl.ds(sid * STR, STR)])
        plsc.subcore_barrier()

        def body(j, carry):
            pltpu.sync_copy(ov, acc.at[sb.at[j]], add=True)
            pltpu.sync_copy(ov, acc.at[ob.at[j]], add=True)
            return carry

        lax.fori_loop(0, NCH, body, 0)
        plsc.subcore_barrier()
        pltpu.sync_copy(acc.at[pl.ds(sid * STR, STR)],
                        out.at[cid, pl.ds(sid * STR, STR)])

    return ck


def _counts_sc(sidx3, oidx3, ones128, zeros128):
    return _counts_builder()(sidx3, oidx3, ones128, zeros128)


def _prep_gconv(p, din):
    (w1, b1), (w2, b2) = p['net1']
    w1T = w1.T
    w2T = w2.T
    (a1, ab1), (a2, ab2) = p['net2']
    return dict(
        a1sT=w1T[:din], a1pT=w1T[din:2 * din], a1oT=w1T[2 * din:],
        b1=b1.reshape(1, HID),
        w2sT=w2T[:, :HID], b2s=b2[:HID].reshape(1, HID),
        w2pT=w2T[:, HID:HID + din], b2p=b2[HID:HID + din].reshape(1, din),
        w2oT=w2T[:, HID + din:], b2o=b2[HID + din:].reshape(1, HID),
        n2aT=a1.T, n2b1=ab1.reshape(1, HID),
        n2bT=a2.T, n2b2=ab2.reshape(1, din),
    )


def kernel(objs, triples, boxes_gt, angles_gt, attributes, obj_to_img, params):
    objs = objs.astype(I32)
    attributes = attributes.astype(I32)
    angles_gt = angles_gt.astype(I32)
    s_idx = triples[:, 0].astype(I32)
    p_idx = triples[:, 1].astype(I32)
    o_idx = triples[:, 2].astype(I32)
    sidx3 = s_idx.reshape(NW, NCH, CB)
    oidx3 = o_idx.reshape(NW, NCH, CB)
    pidx3 = p_idx.reshape(GT, 1, TT)
    objs3 = objs.reshape(GO, 1, TT)
    attrs3 = attributes.reshape(GO, 1, TT)
    angs3 = angles_gt.reshape(GO, 1, TT)
    zeros128 = jnp.zeros((O_PAD, 128), F32)
    ones128 = jnp.ones((CB, 128), F32)

    we = [_prep_gconv(p, 256) for p in params['gconv_ec']]
    wd = [_prep_gconv(p, 128) for p in params['gconv_dc']]

    ove, ovd, attrd, pre1e, pre1d = _prep_call(
        objs3, attrs3, angs3, boxes_gt, params, we[0]['a1pT'], wd[0]['a1pT'])
    for w in we + wd:
        for k2 in ('a1sT', 'a1pT', 'a1oT', 'w2sT', 'w2pT', 'w2oT'):
            w[k2] = w[k2].astype(BF16)
    we[0]['pre1'] = pre1e
    wd[0]['pre1'] = pre1d

    counts = _counts_sc(sidx3, oidx3, ones128, zeros128)

    x = ove
    pv = pidx3
    for l in range(5):
        gs, go = _gather_sc(x, sidx3, oidx3, 128)
        outs = _edge_layer(gs, go, pv, we[l], l == 0, l == 4, 256, True)
        parts = _scatter_sc(tuple(outs[:8]), sidx3, oidx3, zeros128)
        x = _net2_layer(parts, counts, we[l], 256, last=(l == 4))
        pv = outs[8] if l < 4 else None

    mu, logvar = _heads_enc(x, params)

    xd = ovd
    pv = pidx3
    for l in range(5):
        gs, go = _gather_sc(xd, sidx3, oidx3, 128)
        outs = _edge_layer(gs, go, pv, wd[l], l == 0, l == 4, 128, False)
        parts = _scatter_sc(tuple(outs[:8]), sidx3, oidx3, zeros128)
        xd = _net2_layer(parts, counts, wd[l], 128, last=(l == 4))
        pv = outs[8] if l < 4 else None

    boxes_pred, angles_pred = _heads_dec(xd, mu, attrd, params)
    return (mu, logvar, boxes_pred, angles_pred)


# speedup vs baseline: 3.0246x; 1.0215x over previous
import functools

import jax
import jax.numpy as jnp
from jax import lax
from jax.experimental import pallas as pl
from jax.experimental.pallas import tpu as pltpu
from jax.experimental.pallas import tpu_sc as plsc

F32 = jnp.float32
BF16 = jnp.bfloat16
I32 = jnp.int32

O_N = 10000
T_N = 160000
NO = 36
NP = 16
NA = 8
NANG = 24
HID = 512

NC, NS = 2, 16
NW = NC * NS
CB = 128
NCH = 40
PER_W = NCH * CB
T_PAD = NW * PER_W
TB = T_PAD // CB
O_PAD = 10240
STR = O_PAD // NS

TT = 2048
GT = T_PAD // TT
EB = TT // CB
NT = 2000
GO = O_N // NT


def _wspec(a):
    return pl.BlockSpec(a.shape, lambda i: (0,) * a.ndim)


def _espec(d):
    return pl.BlockSpec((TT, d), lambda i: (i, 0))


def _nspec(d):
    return pl.BlockSpec((NT, d), lambda i: (i, 0))


def _idxspec():
    return pl.BlockSpec((1, 1, TT), lambda i: (i, 0, 0))


def _nidxspec():
    return pl.BlockSpec((1, 1, NT), lambda i: (i, 0, 0))


def _onehot(v, n, rows, dt=F32):
    return (v[:, None] == lax.broadcasted_iota(I32, (rows, n), 1)).astype(dt)


def _dot(a, b):
    return jnp.dot(a, b, preferred_element_type=F32)


def _rne16(u):
    return (u + jnp.uint32(0x7FFF) + ((u >> 16) & jnp.uint32(1))) >> 16


def _pack(x):
    d2 = x.shape[1] // 2
    lo = _rne16(lax.bitcast_convert_type(x[:, :d2], jnp.uint32))
    hi = _rne16(lax.bitcast_convert_type(x[:, d2:], jnp.uint32))
    return lax.bitcast_convert_type(lo | (hi << 16), F32)


def _unpack(g):
    w = lax.bitcast_convert_type(g, jnp.uint32)
    lo = lax.bitcast_convert_type(w << 16, F32)
    hi = lax.bitcast_convert_type(w & jnp.uint32(0xFFFF0000), F32)
    return jnp.concatenate([lo, hi], axis=1).astype(BF16)


def _prep_body(objs_r, attrs_r, angs_r, boxes_r, oec, aec, ange, wbT, bb,
               odc, adc, pec, a1pTe, pdc, a1pTd,
               ove_r, ovd_r, attrd_r, pre1e_r, pre1d_r):
    oh_obj = _onehot(objs_r[0, 0], NO, NT)
    oh_att = _onehot(attrs_r[0, 0], NA, NT)
    e_obj = _dot(oh_obj, oec[...])
    e_att = _dot(oh_att, aec[...])
    e_box = _dot(boxes_r[...], wbT[...]) + bb[...]
    e_ang = _dot(_onehot(angs_r[0, 0], NANG, NT), ange[...])
    ove_r[...] = _pack(jnp.concatenate([e_obj, e_att, e_box, e_ang], axis=1))
    d_obj = _dot(oh_obj, odc[...])
    d_att = _dot(oh_att, adc[...])
    ovd_r[...] = jnp.concatenate([d_obj, d_att], axis=1)
    attrd_r[...] = d_att
    pre1e_r[...] = _dot(pec[...], a1pTe[...]).astype(BF16)
    pre1d_r[...] = _dot(pdc[...], a1pTd[...]).astype(BF16)


def _prep_call(objs3, attrs3, angs3, boxes, params, a1pTe, a1pTd):
    wb, bb = params['box_embeddings']
    ins = [objs3, attrs3, angs3, boxes,
           params['obj_emb_ec'], params['attr_emb_ec'], params['angle_emb'],
           wb.T, bb.reshape(1, -1),
           params['obj_emb_dc'], params['attr_emb_dc'],
           params['pred_emb_ec'], a1pTe, params['pred_emb_dc'], a1pTd]
    specs = [_nidxspec(), _nidxspec(), _nidxspec(),
             pl.BlockSpec((NT, 6), lambda i: (i, 0))] + [_wspec(x) for x in ins[4:]]
    out_shape = [jax.ShapeDtypeStruct((O_N, 128), F32),
                 jax.ShapeDtypeStruct((O_N, 128), F32),
                 jax.ShapeDtypeStruct((O_N, 32), F32),
                 jax.ShapeDtypeStruct((NP, HID), BF16),
                 jax.ShapeDtypeStruct((NP, HID), BF16)]
    out_specs = [_nspec(128), _nspec(128), _nspec(32),
                 pl.BlockSpec((NP, HID), lambda i: (0, 0)),
                 pl.BlockSpec((NP, HID), lambda i: (0, 0))]
    return pl.pallas_call(_prep_body, grid=(GO,), in_specs=specs,
                          out_specs=out_specs, out_shape=out_shape)(*ins)


def _edge_body(layer1, last, packed):
    def body(*refs):
        if layer1:
            pv, gs, go, pmat, a1sT, a1oT, b1, w2sT, b2s, w2oT, b2o, *rest = refs
        else:
            pv, gs, go, pmat, a1sT, a1oT, b1, w2sT, b2s, w2oT, b2o, *rest = refs
        if not last:
            w2pT, b2p, *rest = rest
        outs = rest
        if packed:
            gsm = _unpack(gs[...].reshape(TT, gs.shape[2]))
            gom = _unpack(go[...].reshape(TT, go.shape[2]))
        else:
            gsm = gs[...].reshape(TT, gs.shape[2]).astype(BF16)
            gom = go[...].reshape(TT, go.shape[2]).astype(BF16)
        h = _dot(gsm, a1sT[...]) + _dot(gom, a1oT[...])
        if layer1:
            h += _dot(_onehot(pv[0, 0], NP, TT, BF16), pmat[...])
        else:
            h += _dot(pv[...], pmat[...])
        hb = jnp.maximum(h + b1[...], 0.0).astype(BF16)
        s = jnp.maximum(_dot(hb, w2sT[...]) + b2s[...], 0.0)
        o = jnp.maximum(_dot(hb, w2oT[...]) + b2o[...], 0.0)
        for k in range(4):
            outs[k][...] = s[:, k * 128:(k + 1) * 128].reshape(EB, CB, 128)
            outs[4 + k][...] = o[:, k * 128:(k + 1) * 128].reshape(EB, CB, 128)
        if not last:
            outs[8][...] = jnp.maximum(
                _dot(hb, w2pT[...]) + b2p[...], 0.0).astype(BF16)
    return body


def _gspec(d):
    return pl.BlockSpec((EB, CB, d), lambda i: (i, 0, 0))


def _edge_layer(gs, go, pvp, w, layer1, last, din, packed):
    d2 = din // 2 if packed else din
    if layer1:
        ins = [pvp, gs, go, w['pre1']]
        specs = [_idxspec(), _gspec(d2), _gspec(d2), _wspec(w['pre1'])]
    else:
        ins = [pvp, gs, go, w['a1pT']]
        specs = [_espec(din), _gspec(d2), _gspec(d2), _wspec(w['a1pT'])]
    tail = [w['a1sT'], w['a1oT'], w['b1'], w['w2sT'], w['b2s'], w['w2oT'], w['b2o']]
    if not last:
        tail += [w['w2pT'], w['b2p']]
    ins += tail
    specs += [_wspec(x) for x in tail]
    out_shape = [jax.ShapeDtypeStruct((TB, CB, 128), F32) for _ in range(8)]
    out_specs = [_gspec(128) for _ in range(8)]
    if not last:
        out_shape.append(jax.ShapeDtypeStruct((T_PAD, din), BF16))
        out_specs.append(_espec(din))
    return pl.pallas_call(_edge_body(layer1, last, packed), grid=(GT,), in_specs=specs,
                          out_specs=out_specs, out_shape=out_shape)(*ins)


def _net2_body(p_r, c_r, aT, ab, bT, bb, out_r):
    cnt = c_r[0, :, 0:1] + c_r[1, :, 0:1]
    inv = 1.0 / jnp.maximum(cnt, 1.0)
    aTm = aT[...]
    h = jnp.zeros((NT, HID), F32) + ab[...]
    for k in range(4):
        pk = (p_r[0, k] + p_r[1, k]) * inv
        h += _dot(pk, aTm[k * 128:(k + 1) * 128])
    h = jnp.maximum(h, 0.0)
    res = jnp.maximum(_dot(h, bT[...]) + bb[...], 0.0)
    if out_r.shape[1] * 2 == bT.shape[1]:
        out_r[...] = _pack(res)
    else:
        out_r[...] = res


def _net2_layer(parts, counts, w, dn, last=False):
    ins = [parts, counts, w['n2aT'], w['n2b1'], w['n2bT'], w['n2b2']]
    specs = [pl.BlockSpec((NC, 4, NT, 128), lambda i: (0, 0, i, 0)),
             pl.BlockSpec((NC, NT, 128), lambda i: (0, i, 0)),
             _wspec(w['n2aT']), _wspec(w['n2b1']),
             _wspec(w['n2bT']), _wspec(w['n2b2'])]
    dn_out = dn // 2 if (not last and dn == 256) else dn
    return pl.pallas_call(_net2_body, grid=(GO,), in_specs=specs,
                          out_specs=_nspec(dn_out),
                          out_shape=jax.ShapeDtypeStruct((O_N, dn_out), F32))(*ins)


def _heads_enc_body(x_r, m1T, m1b, m2T, m2b, bmT, bmb, bvT, bvb,
                    a1T, a1b, a2T, a2b, amT, amb, avT, avb, mu_r, lv_r):
    x = x_r[...]
    ovb = jnp.maximum(_dot(x, m1T[...]) + m1b[...], 0.0)
    ovb = jnp.maximum(_dot(ovb, m2T[...]) + m2b[...], 0.0)
    mub = _dot(ovb, bmT[...]) + bmb[...]
    lvb = _dot(ovb, bvT[...]) + bvb[...]
    ova = jnp.maximum(_dot(x, a1T[...]) + a1b[...], 0.0)
    ova = jnp.maximum(_dot(ova, a2T[...]) + a2b[...], 0.0)
    mua = _dot(ova, amT[...]) + amb[...]
    lva = _dot(ova, avT[...]) + avb[...]
    mu_r[...] = jnp.concatenate([mub, mua], axis=1)
    lv_r[...] = jnp.concatenate([lvb, lva], axis=1)


def _heads_enc(x, params):
    def tb(layers):
        out = []
        for wgt, b in layers:
            out += [wgt.T, b.reshape(1, -1)]
        return out
    ins = ([x] + tb(params['box_mean_var']) + tb(params['box_mean'])
           + tb(params['box_var']) + tb(params['angle_mean_var'])
           + tb(params['angle_mean']) + tb(params['angle_var']))
    specs = [_nspec(256)] + [_wspec(a) for a in ins[1:]]
    out_shape = [jax.ShapeDtypeStruct((O_N, 128), F32)] * 2
    out_specs = [_nspec(128)] * 2
    return pl.pallas_call(_heads_enc_body, grid=(GO,), in_specs=specs,
                          out_specs=out_specs, out_shape=out_shape)(*ins)


def _heads_dec_body(xd_r, z_r, at_r, b1T, b1b, b2T, b2b, n1T, n1b, n2T, n2b,
                    box_r, ang_r):
    xd = xd_r[...]
    z = z_r[...]
    xb = jnp.concatenate([xd, z, at_r[...]], axis=1)
    hb = jnp.maximum(_dot(xb, b1T[...]) + b1b[...], 0.0)
    box_r[...] = _dot(hb, b2T[...]) + b2b[...]
    xa = jnp.concatenate([xd, z], axis=1)
    ha = jnp.maximum(_dot(xa, n1T[...]) + n1b[...], 0.0)
    aa = _dot(ha, n2T[...]) + n2b[...]
    m = jnp.max(aa, axis=1, keepdims=True)
    ex = jnp.exp(aa - m)
    ang_r[...] = (aa - m) - jnp.log(jnp.sum(ex, axis=1, keepdims=True))


def _heads_dec(xd, z, attrd, params):
    def tb(layers):
        out = []
        for wgt, b in layers:
            out += [wgt.T, b.reshape(1, -1)]
        return out
    ins = [xd, z, attrd] + tb(params['box_net']) + tb(params['angle_net'])
    specs = [_nspec(128), _nspec(128), _nspec(32)] + [_wspec(a) for a in ins[3:]]
    out_shape = [jax.ShapeDtypeStruct((O_N, 6), F32),
                 jax.ShapeDtypeStruct((O_N, NANG), F32)]
    out_specs = [pl.BlockSpec((NT, 6), lambda i: (i, 0)),
                 pl.BlockSpec((NT, NANG), lambda i: (i, 0))]
    return pl.pallas_call(_heads_dec_body, grid=(GO,), in_specs=specs,
                          out_specs=out_specs, out_shape=out_shape)(*ins)


@functools.lru_cache(None)
def _gather_builder(d):
    mesh = plsc.VectorSubcoreMesh(core_axis_name="c", subcore_axis_name="s")

    @functools.partial(
        pl.kernel,
        out_type=(jax.ShapeDtypeStruct((TB, CB, d), F32),
                  jax.ShapeDtypeStruct((TB, CB, d), F32)),
        mesh=mesh,
        scratch_types=[pltpu.VMEM((NCH, CB), I32), pltpu.VMEM((NCH, CB), I32),
                       pltpu.VMEM((1, CB, d), F32), pltpu.VMEM((1, CB, d), F32),
                       pltpu.SemaphoreType.DMA, pltpu.SemaphoreType.DMA])
    def gk(table, sidx, oidx, gs_out, go_out, sb, ob, ra, rb, sema, semb):
        cid = lax.axis_index("c")
        sid = lax.axis_index("s")
        wid = sid * NC + cid
        wblk = wid * NCH
        pltpu.sync_copy(sidx.at[wid], sb)
        pltpu.sync_copy(oidx.at[wid], ob)

        def phase(ib, out):
            pltpu.async_copy(table.at[ib.at[0]], ra.at[0], sema)

            def body2(cc, carry):
                c = cc * 2
                pltpu.async_copy(table.at[ib.at[c + 1]], rb.at[0], semb)
                pltpu.make_async_copy(table.at[ib.at[0]], ra.at[0], sema).wait()
                pltpu.sync_copy(ra, out.at[pl.ds(wblk + c, 1)])

                @pl.when(c + 2 < NCH)
                def _():
                    pltpu.async_copy(table.at[ib.at[c + 2]], ra.at[0], sema)

                pltpu.make_async_copy(table.at[ib.at[0]], rb.at[0], semb).wait()
                pltpu.sync_copy(rb, out.at[pl.ds(wblk + c + 1, 1)])
                return carry

            lax.fori_loop(0, NCH // 2, body2, 0)

        phase(sb, gs_out)
        phase(ob, go_out)

    return gk


def _gather_sc(table, sidx3, oidx3, d):
    return _gather_builder(d)(table, sidx3, oidx3)


@functools.lru_cache(None)
def _scatter_builder():
    mesh = plsc.VectorSubcoreMesh(core_axis_name="c", subcore_axis_name="s")

    @functools.partial(
        pl.kernel,
        out_type=jax.ShapeDtypeStruct((NC, 4, O_PAD, 128), F32),
        mesh=mesh,
        scratch_types=[pltpu.VMEM((NCH, CB), I32), pltpu.VMEM((NCH, CB), I32),
                       pltpu.VMEM((1, CB, 128), F32),
                       pltpu.VMEM((1, CB, 128), F32),
                       pltpu.VMEM_SHARED((O_PAD, 128), F32),
                       pltpu.SemaphoreType.DMA, pltpu.SemaphoreType.DMA])
    def sk(s0, s1, s2, s3, o0, o1, o2, o3, sidx, oidx, zeros, out,
           sb, ob, va, vb, acc, m1, m2):
        cid = lax.axis_index("c")
        sid = lax.axis_index("s")
        wid = sid * NC + cid
        wblk = wid * NCH
        pltpu.sync_copy(sidx.at[wid], sb)
        pltpu.sync_copy(oidx.at[wid], ob)
        svals = (s0, s1, s2, s3)
        ovals = (o0, o1, o2, o3)
        for c4 in range(4):
            pltpu.sync_copy(zeros.at[pl.ds(sid * STR, STR)],
                            acc.at[pl.ds(sid * STR, STR)])
            plsc.subcore_barrier()
            sv = svals[c4]
            ov = ovals[c4]
            pltpu.async_copy(sv.at[pl.ds(wblk, 1)], va, m1)
            pltpu.async_copy(ov.at[pl.ds(wblk, 1)], vb, m2)

            def body(j, carry):
                pltpu.make_async_copy(sv.at[pl.ds(wblk, 1)], va, m1).wait()
                pltpu.sync_copy(va.at[0], acc.at[sb.at[j]], add=True)

                @pl.when(j + 1 < NCH)
                def _():
                    pltpu.async_copy(sv.at[pl.ds(wblk + j + 1, 1)], va, m1)

                pltpu.make_async_copy(ov.at[pl.ds(wblk, 1)], vb, m2).wait()
                pltpu.sync_copy(vb.at[0], acc.at[ob.at[j]], add=True)

                @pl.when(j + 1 < NCH)
                def _():
                    pltpu.async_copy(ov.at[pl.ds(wblk + j + 1, 1)], vb, m2)

                return carry

            lax.fori_loop(0, NCH, body, 0)
            plsc.subcore_barrier()
            pltpu.sync_copy(acc.at[pl.ds(sid * STR, STR)],
                            out.at[cid, c4, pl.ds(sid * STR, STR)])

    return sk


def _scatter_sc(so_chunks, sidx3, oidx3, zeros128):
    return _scatter_builder()(*so_chunks, sidx3, oidx3, zeros128)


@functools.lru_cache(None)
def _counts_builder():
    mesh = plsc.VectorSubcoreMesh(core_axis_name="c", subcore_axis_name="s")

    @functools.partial(
        pl.kernel,
        out_type=jax.ShapeDtypeStruct((NC, O_PAD, 128), F32),
        mesh=mesh,
        scratch_types=[pltpu.VMEM((NCH, CB), I32), pltpu.VMEM((NCH, CB), I32),
                       pltpu.VMEM((CB, 128), F32),
                       pltpu.VMEM_SHARED((O_PAD, 128), F32)])
    def ck(sidx, oidx, ones, zeros, out, sb, ob, ov, acc):
        cid = lax.axis_index("c")
        sid = lax.axis_index("s")
        wid = sid * NC + cid
        pltpu.sync_copy(sidx.at[wid], sb)
        pltpu.sync_copy(oidx.at[wid], ob)
        pltpu.sync_copy(ones, ov)
        pltpu.sync_copy(zeros.at[pl.ds(sid * STR, STR)],
                        acc.at[pl.ds(sid * STR, STR)])
        plsc.subcore_barrier()

        def body(j, carry):
            pltpu.sync_copy(ov, acc.at[sb.at[j]], add=True)
            pltpu.sync_copy(ov, acc.at[ob.at[j]], add=True)
            return carry

        lax.fori_loop(0, NCH, body, 0)
        plsc.subcore_barrier()
        pltpu.sync_copy(acc.at[pl.ds(sid * STR, STR)],
                        out.at[cid, pl.ds(sid * STR, STR)])

    return ck


def _counts_sc(sidx3, oidx3, ones128, zeros128):
    return _counts_builder()(sidx3, oidx3, ones128, zeros128)


def _prep_gconv(p, din):
    (w1, b1), (w2, b2) = p['net1']
    w1T = w1.T
    w2T = w2.T
    (a1, ab1), (a2, ab2) = p['net2']
    return dict(
        a1sT=w1T[:din], a1pT=w1T[din:2 * din], a1oT=w1T[2 * din:],
        b1=b1.reshape(1, HID),
        w2sT=w2T[:, :HID], b2s=b2[:HID].reshape(1, HID),
        w2pT=w2T[:, HID:HID + din], b2p=b2[HID:HID + din].reshape(1, din),
        w2oT=w2T[:, HID + din:], b2o=b2[HID + din:].reshape(1, HID),
        n2aT=a1.T, n2b1=ab1.reshape(1, HID),
        n2bT=a2.T, n2b2=ab2.reshape(1, din),
    )


def kernel(objs, triples, boxes_gt, angles_gt, attributes, obj_to_img, params):
    objs = objs.astype(I32)
    attributes = attributes.astype(I32)
    angles_gt = angles_gt.astype(I32)
    s_idx = triples[:, 0].astype(I32)
    p_idx = triples[:, 1].astype(I32)
    o_idx = triples[:, 2].astype(I32)
    npad = T_PAD - s_idx.shape[0]
    pad0 = jnp.zeros((npad,), I32)
    padd = jnp.full((npad,), O_N, I32)
    sidx3g = jnp.concatenate([s_idx, pad0]).reshape(NW, NCH, CB)
    oidx3g = jnp.concatenate([o_idx, pad0]).reshape(NW, NCH, CB)
    sidx3s = jnp.concatenate([s_idx, padd]).reshape(NW, NCH, CB)
    oidx3s = jnp.concatenate([o_idx, padd]).reshape(NW, NCH, CB)
    pidx3 = jnp.concatenate([p_idx, pad0]).reshape(GT, 1, TT)
    objs3 = objs.reshape(GO, 1, NT)
    attrs3 = attributes.reshape(GO, 1, NT)
    angs3 = angles_gt.reshape(GO, 1, NT)
    zeros128 = jnp.zeros((O_PAD, 128), F32)
    ones128 = jnp.ones((CB, 128), F32)

    we = [_prep_gconv(p, 256) for p in params['gconv_ec']]
    wd = [_prep_gconv(p, 128) for p in params['gconv_dc']]

    ove, ovd, attrd, pre1e, pre1d = _prep_call(
        objs3, attrs3, angs3, boxes_gt, params, we[0]['a1pT'], wd[0]['a1pT'])
    for w in we + wd:
        for k2 in ('a1sT', 'a1pT', 'a1oT', 'w2sT', 'w2pT', 'w2oT'):
            w[k2] = w[k2].astype(BF16)
    we[0]['pre1'] = pre1e
    wd[0]['pre1'] = pre1d

    counts = _counts_sc(sidx3s, oidx3s, ones128, zeros128)

    x = ove
    pv = pidx3
    for l in range(5):
        gs, go = _gather_sc(x, sidx3g, oidx3g, 128)
        outs = _edge_layer(gs, go, pv, we[l], l == 0, l == 4, 256, True)
        parts = _scatter_sc(tuple(outs[:8]), sidx3s, oidx3s, zeros128)
        x = _net2_layer(parts, counts, we[l], 256, last=(l == 4))
        pv = outs[8] if l < 4 else None

    mu, logvar = _heads_enc(x, params)

    xd = ovd
    pv = pidx3
    for l in range(5):
        gs, go = _gather_sc(xd, sidx3g, oidx3g, 128)
        outs = _edge_layer(gs, go, pv, wd[l], l == 0, l == 4, 128, False)
        parts = _scatter_sc(tuple(outs[:8]), sidx3s, oidx3s, zeros128)
        xd = _net2_layer(parts, counts, wd[l], 128, last=(l == 4))
        pv = outs[8] if l < 4 else None

    boxes_pred, angles_pred = _heads_dec(xd, mu, attrd, params)
    return (mu, logvar, boxes_pred, angles_pred)
